# baseline (speedup 1.0000x reference)
import functools

import numpy as np
import jax
import jax.numpy as jnp
from jax import lax
from jax.experimental import pallas as pl
from jax.experimental.pallas import tpu as pltpu
from jax.experimental.pallas import tpu_sc as plsc

N = 10000
E = 320000
D_IN = 128
DH = 64
HF = 32
DOUT = 40
L = 16
ALPHA = 0.1
THETA = 0.5

NC = 2
NS = 16
NW = NC * NS
NPAD = 10240
RPS = NPAD // NS
K = 128
CPW = 80
E_PAD = NW * CPW * K
PAD_NODE = N


NB = 8
GA = 4


def _sc_propagate_body(hs0_hbm, hs1_hbm, row_hbm, col_hbm, o0_hbm, o1_hbm,
                       ridx, cidx, rows0, rows1, rows2, rows3, rows4, rows5,
                       rows6, rows7, zbuf,
                       agg_sh, hs_sh,
                       gsem0, gsem1, gsem2, gsem3, gsem4, gsem5, gsem6, gsem7,
                       ssem0, ssem1, ssem2, ssem3, ssem4, ssem5, ssem6, ssem7):
    c = lax.axis_index("c")
    s = lax.axis_index("s")
    w = c * NS + s
    my_rows = pl.ds(s * RPS, RPS)
    rows = (rows0, rows1, rows2, rows3, rows4, rows5, rows6, rows7)
    gsem = (gsem0, gsem1, gsem2, gsem3, gsem4, gsem5, gsem6, gsem7)
    ssem = (ssem0, ssem1, ssem2, ssem3, ssem4, ssem5, ssem6, ssem7)

    ri_cp = pltpu.make_async_copy(row_hbm.at[w], ridx, gsem0)
    ci_cp = pltpu.make_async_copy(col_hbm.at[w], cidx, gsem1)
    ri_cp.start()
    ci_cp.start()

    zero16 = jnp.zeros((16,), jnp.float32)

    def zrow(i, carry):
        for jj in range(HF // 16):
            zbuf[i, pl.ds(jj * 16, 16)] = zero16
        return carry

    lax.fori_loop(0, RPS, zrow, 0)
    ri_cp.wait()
    ci_cp.wait()

    def g_cp(j, t):
        return pltpu.make_async_copy(hs_sh.at[ridx.at[j]], rows[t], gsem[t])

    def s_start(j, t):
        pltpu.async_copy(rows[t], agg_sh.at[cidx.at[j]], ssem[t], add=True)

    def s_wait(j, t):
        pltpu.make_async_copy(rows[t], agg_sh.at[cidx.at[j]], ssem[t]).wait()

    for hs_hbm, o_hbm in ((hs0_hbm, o0_hbm), (hs1_hbm, o1_hbm)):
        pltpu.sync_copy(hs_hbm.at[my_rows], hs_sh.at[my_rows])
        pltpu.sync_copy(zbuf, agg_sh.at[my_rows])
        plsc.subcore_barrier()

        def quad(q, carry):
            i0 = q * NB
            for t in range(NB):
                i = i0 + t

                @pl.when(jnp.logical_and(i >= NB, i - NB < CPW))
                def _():
                    s_wait(i - NB, t)

                @pl.when(i < CPW)
                def _():
                    g_cp(i, t).start()

                @pl.when(jnp.logical_and(i >= GA, i - GA < CPW))
                def _():
                    g_cp(i - GA, (t - GA) % NB).wait()
                    s_start(i - GA, (t - GA) % NB)

            return carry

        lax.fori_loop(0, (CPW + NB) // NB, quad, 0)
        plsc.subcore_barrier()

        pltpu.sync_copy(agg_sh.at[my_rows], o_hbm.at[c, my_rows])


@functools.cache
def _get_sc_propagate():
    return pl.kernel(
        _sc_propagate_body,
        out_type=[
            jax.ShapeDtypeStruct((NC, NPAD, HF), jnp.float32),
            jax.ShapeDtypeStruct((NC, NPAD, HF), jnp.float32),
        ],
        mesh=plsc.VectorSubcoreMesh(core_axis_name="c", subcore_axis_name="s",
                                    num_cores=NC, num_subcores=NS),
        scratch_types=[
            pltpu.VMEM((CPW, K), jnp.int32),
            pltpu.VMEM((CPW, K), jnp.int32),
            pltpu.VMEM((K, HF), jnp.float32),
            pltpu.VMEM((K, HF), jnp.float32),
            pltpu.VMEM((K, HF), jnp.float32),
            pltpu.VMEM((K, HF), jnp.float32),
            pltpu.VMEM((K, HF), jnp.float32),
            pltpu.VMEM((K, HF), jnp.float32),
            pltpu.VMEM((K, HF), jnp.float32),
            pltpu.VMEM((K, HF), jnp.float32),
            pltpu.VMEM((RPS, HF), jnp.float32),
            pltpu.VMEM_SHARED((NPAD, HF), jnp.float32),
            pltpu.VMEM_SHARED((NPAD, HF), jnp.float32),
            pltpu.SemaphoreType.DMA,
            pltpu.SemaphoreType.DMA,
            pltpu.SemaphoreType.DMA,
            pltpu.SemaphoreType.DMA,
            pltpu.SemaphoreType.DMA,
            pltpu.SemaphoreType.DMA,
            pltpu.SemaphoreType.DMA,
            pltpu.SemaphoreType.DMA,
            pltpu.SemaphoreType.DMA,
            pltpu.SemaphoreType.DMA,
            pltpu.SemaphoreType.DMA,
            pltpu.SemaphoreType.DMA,
            pltpu.SemaphoreType.DMA,
            pltpu.SemaphoreType.DMA,
            pltpu.SemaphoreType.DMA,
            pltpu.SemaphoreType.DMA,
        ],
        compiler_params=pltpu.CompilerParams(use_tc_tiling_on_sc=False),
    )


BN = 1024
GRID = NPAD // BN


def _tc_init_body(x_ref, win_ref, bin_ref, d0_ref, h_ref, hs0_ref, hs1_ref,
                  dinv_ref):
    xb = x_ref[...]
    h = jnp.maximum(
        jnp.dot(xb, win_ref[...], preferred_element_type=jnp.float32)
        + bin_ref[...], 0.0)
    deg = d0_ref[0, :, 0:1] + d0_ref[1, :, 0:1] + 1.0
    dinv = lax.rsqrt(deg)
    hs = h * dinv
    h_ref[...] = h
    hs0_ref[...] = hs[:, :HF]
    hs1_ref[...] = hs[:, HF:]
    dinv_ref[...] = dinv


def _make_tc_init(interpret=False):
    return pl.pallas_call(
        _tc_init_body,
        grid=(GRID,),
        in_specs=[
            pl.BlockSpec((BN, D_IN), lambda i: (i, 0)),
            pl.BlockSpec((D_IN, DH), lambda i: (0, 0)),
            pl.BlockSpec((1, DH), lambda i: (0, 0)),
            pl.BlockSpec((NC, BN, HF), lambda i: (0, i, 0)),
        ],
        out_specs=[
            pl.BlockSpec((BN, DH), lambda i: (i, 0)),
            pl.BlockSpec((BN, HF), lambda i: (i, 0)),
            pl.BlockSpec((BN, HF), lambda i: (i, 0)),
            pl.BlockSpec((BN, 1), lambda i: (i, 0)),
        ],
        out_shape=[
            jax.ShapeDtypeStruct((NPAD, DH), jnp.float32),
            jax.ShapeDtypeStruct((NPAD, HF), jnp.float32),
            jax.ShapeDtypeStruct((NPAD, HF), jnp.float32),
            jax.ShapeDtypeStruct((NPAD, 1), jnp.float32),
        ],
        interpret=interpret,
    )


def _tc_layer_body(a0_ref, a1_ref, h_ref, h0_ref, dinv_ref, w_ref, g_ref,
                   b_ref, ho_ref, hs0_ref, hs1_ref, *, beta):
    dinv = dinv_ref[...]
    raw = jnp.concatenate([a0_ref[0] + a0_ref[1], a1_ref[0] + a1_ref[1]],
                          axis=1)
    h = h_ref[...]
    agg = dinv * raw + (dinv * dinv) * h
    z = (1.0 - ALPHA) * agg + ALPHA * h0_ref[...]
    z = (1.0 - beta) * z + beta * jnp.dot(
        z, w_ref[0], preferred_element_type=jnp.float32)
    a = jnp.maximum(z, 0.0) + h
    mu = jnp.mean(a, axis=1, keepdims=True)
    var = jnp.mean((a - mu) ** 2, axis=1, keepdims=True)
    hn = (a - mu) * lax.rsqrt(var + 1e-5) * g_ref[0] + b_ref[0]
    hs = hn * dinv
    ho_ref[...] = hn
    hs0_ref[...] = hs[:, :HF]
    hs1_ref[...] = hs[:, HF:]


def _make_tc_layer(li, beta, interpret=False):
    return pl.pallas_call(
        functools.partial(_tc_layer_body, beta=beta),
        grid=(GRID,),
        in_specs=[
            pl.BlockSpec((NC, BN, HF), lambda i: (0, i, 0)),
            pl.BlockSpec((NC, BN, HF), lambda i: (0, i, 0)),
            pl.BlockSpec((BN, DH), lambda i: (i, 0)),
            pl.BlockSpec((BN, DH), lambda i: (i, 0)),
            pl.BlockSpec((BN, 1), lambda i: (i, 0)),
            pl.BlockSpec((1, DH, DH), lambda i, li=li: (li, 0, 0)),
            pl.BlockSpec((1, 1, DH), lambda i, li=li: (li, 0, 0)),
            pl.BlockSpec((1, 1, DH), lambda i, li=li: (li, 0, 0)),
        ],
        out_specs=[
            pl.BlockSpec((BN, DH), lambda i: (i, 0)),
            pl.BlockSpec((BN, HF), lambda i: (i, 0)),
            pl.BlockSpec((BN, HF), lambda i: (i, 0)),
        ],
        out_shape=[
            jax.ShapeDtypeStruct((NPAD, DH), jnp.float32),
            jax.ShapeDtypeStruct((NPAD, HF), jnp.float32),
            jax.ShapeDtypeStruct((NPAD, HF), jnp.float32),
        ],
        interpret=interpret,
    )


def _tc_out_body(h_ref, wout_ref, bout_ref, o_ref):
    o_ref[...] = jnp.dot(
        h_ref[...], wout_ref[...], preferred_element_type=jnp.float32
    ) + bout_ref[...]


def _make_tc_out(interpret=False):
    return pl.pallas_call(
        _tc_out_body,
        grid=(GRID,),
        in_specs=[
            pl.BlockSpec((BN, DH), lambda i: (i, 0)),
            pl.BlockSpec((DH, DOUT), lambda i: (0, 0)),
            pl.BlockSpec((1, DOUT), lambda i: (0, 0)),
        ],
        out_specs=pl.BlockSpec((BN, DOUT), lambda i: (i, 0)),
        out_shape=jax.ShapeDtypeStruct((NPAD, DOUT), jnp.float32),
        interpret=interpret,
    )


_tc_init = _make_tc_init()
_tc_layers = [
    _make_tc_layer(i, float(np.log(THETA / (i + 1) + 1.0))) for i in range(L)
]
_tc_out = _make_tc_out()


def kernel(x, edge_index, W_in, b_in, W_conv, ln_g, ln_b, W_out, b_out):
    row = edge_index[0]
    col = edge_index[1]
    pad = jnp.full((E_PAD - E,), PAD_NODE, jnp.int32)
    row2 = jnp.concatenate([row, pad]).reshape(NW, CPW, K)
    col2 = jnp.concatenate([col, pad]).reshape(NW, CPW, K)

    sc_propagate = _get_sc_propagate()
    ones_half = jnp.ones((NPAD, HF), jnp.float32)
    deg2, _ = sc_propagate(ones_half, ones_half, row2, col2)

    x_pad = jnp.pad(x, ((0, NPAD - N), (0, 0)))
    h, hs0, hs1, dinv = _tc_init(x_pad, W_in, b_in.reshape(1, DH), deg2)
    h0 = h
    ln_g3 = ln_g.reshape(L, 1, DH)
    ln_b3 = ln_b.reshape(L, 1, DH)
    for i in range(L):
        a0, a1 = sc_propagate(hs0, hs1, row2, col2)
        h, hs0, hs1 = _tc_layers[i](a0, a1, h, h0, dinv, W_conv, ln_g3, ln_b3)
    out = _tc_out(h, W_out, b_out.reshape(1, DOUT))
    return out[:N]


# speedup vs baseline: 1.0331x; 1.0331x over previous
import functools

import numpy as np
import jax
import jax.numpy as jnp
from jax import lax
from jax.experimental import pallas as pl
from jax.experimental.pallas import tpu as pltpu
from jax.experimental.pallas import tpu_sc as plsc

N = 10000
E = 320000
D_IN = 128
DH = 64
HF = 32
DOUT = 40
L = 16
ALPHA = 0.1
THETA = 0.5

NC = 2
NS = 16
NW = NC * NS
NPAD = 10240
RPS = NPAD // NS
K = 128
CPW = 80
E_PAD = NW * CPW * K
PAD_NODE = N


NB = 4


def _sc_propagate_body(hs0_hbm, hs1_hbm, row_hbm, col_hbm, o0_hbm, o1_hbm,
                       ridx, cidx, rows0, rows1, rows2, rows3, zbuf,
                       agg_sh, hs_sh,
                       gsem0, gsem1, gsem2, gsem3,
                       ssem0, ssem1, ssem2, ssem3):
    c = lax.axis_index("c")
    s = lax.axis_index("s")
    w = c * NS + s
    my_rows = pl.ds(s * RPS, RPS)
    rows = (rows0, rows1, rows2, rows3)
    gsem = (gsem0, gsem1, gsem2, gsem3)
    ssem = (ssem0, ssem1, ssem2, ssem3)

    ri_cp = pltpu.make_async_copy(row_hbm.at[w], ridx, gsem0)
    ci_cp = pltpu.make_async_copy(col_hbm.at[w], cidx, gsem1)
    ri_cp.start()
    ci_cp.start()

    zero16 = jnp.zeros((16,), jnp.float32)

    def zrow(i, carry):
        for jj in range(HF // 16):
            zbuf[i, pl.ds(jj * 16, 16)] = zero16
        return carry

    lax.fori_loop(0, RPS, zrow, 0)
    ri_cp.wait()
    ci_cp.wait()

    def g_cp(j, t):
        return pltpu.make_async_copy(hs_sh.at[ridx.at[j]], rows[t], gsem[t])

    def s_start(j, t):
        pltpu.async_copy(rows[t], agg_sh.at[cidx.at[j]], ssem[t], add=True)

    def s_wait(j, t):
        pltpu.make_async_copy(rows[t], agg_sh.at[cidx.at[j]], ssem[t]).wait()

    for hs_hbm, o_hbm in ((hs0_hbm, o0_hbm), (hs1_hbm, o1_hbm)):
        pltpu.sync_copy(hs_hbm.at[my_rows], hs_sh.at[my_rows])
        pltpu.sync_copy(zbuf, agg_sh.at[my_rows])
        plsc.subcore_barrier()

        def quad(q, carry):
            i0 = q * NB
            for t in range(NB):
                i = i0 + t

                @pl.when(jnp.logical_and(i >= NB, i - NB < CPW))
                def _():
                    s_wait(i - NB, t)

                @pl.when(i < CPW)
                def _():
                    g_cp(i, t).start()

                @pl.when(jnp.logical_and(i >= 2, i - 2 < CPW))
                def _():
                    g_cp(i - 2, (t - 2) % NB).wait()
                    s_start(i - 2, (t - 2) % NB)

            return carry

        lax.fori_loop(0, (CPW + NB) // NB, quad, 0)
        plsc.subcore_barrier()

        pltpu.sync_copy(agg_sh.at[my_rows], o_hbm.at[c, my_rows])


def _sc_deg_body(col_hbm, out_hbm, cidx, ones_v, zbuf, agg_sh, ss0, ss1):
    c = lax.axis_index("c")
    s = lax.axis_index("s")
    w = c * NS + s
    my_rows = pl.ds(s * RPS, RPS)

    ci_cp = pltpu.make_async_copy(col_hbm.at[w], cidx, ss0)
    ci_cp.start()

    zero16 = jnp.zeros((16,), jnp.float32)
    one16 = jnp.ones((16,), jnp.float32)

    def zrow(i, carry):
        for jj in range(HF // 16):
            zbuf[i, pl.ds(jj * 16, 16)] = zero16
        return carry

    lax.fori_loop(0, RPS, zrow, 0)

    def orow(i, carry):
        for jj in range(HF // 16):
            ones_v[i, pl.ds(jj * 16, 16)] = one16
        return carry

    lax.fori_loop(0, K, orow, 0)
    pltpu.sync_copy(zbuf, agg_sh.at[my_rows])
    ci_cp.wait()
    plsc.subcore_barrier()

    def s_cp(j, sem):
        return pltpu.make_async_copy(ones_v, agg_sh.at[cidx.at[j]], sem)

    def pair(p, carry):
        j = 2 * p

        @pl.when(p > 0)
        def _():
            s_cp(j - 2, ss0).wait()
            s_cp(j - 1, ss1).wait()

        pltpu.async_copy(ones_v, agg_sh.at[cidx.at[j]], ss0, add=True)
        pltpu.async_copy(ones_v, agg_sh.at[cidx.at[j + 1]], ss1, add=True)
        return carry

    lax.fori_loop(0, CPW // 2, pair, 0)
    s_cp(CPW - 2, ss0).wait()
    s_cp(CPW - 1, ss1).wait()
    plsc.subcore_barrier()
    pltpu.sync_copy(agg_sh.at[my_rows], out_hbm.at[c, my_rows])


@functools.cache
def _get_sc_deg():
    return pl.kernel(
        _sc_deg_body,
        out_type=jax.ShapeDtypeStruct((NC, NPAD, HF), jnp.float32),
        mesh=plsc.VectorSubcoreMesh(core_axis_name="c", subcore_axis_name="s",
                                    num_cores=NC, num_subcores=NS),
        scratch_types=[
            pltpu.VMEM((CPW, K), jnp.int32),
            pltpu.VMEM((K, HF), jnp.float32),
            pltpu.VMEM((RPS, HF), jnp.float32),
            pltpu.VMEM_SHARED((NPAD, HF), jnp.float32),
            pltpu.SemaphoreType.DMA,
            pltpu.SemaphoreType.DMA,
        ],
        compiler_params=pltpu.CompilerParams(use_tc_tiling_on_sc=False),
    )


@functools.cache
def _get_sc_propagate():
    return pl.kernel(
        _sc_propagate_body,
        out_type=[
            jax.ShapeDtypeStruct((NC, NPAD, HF), jnp.float32),
            jax.ShapeDtypeStruct((NC, NPAD, HF), jnp.float32),
        ],
        mesh=plsc.VectorSubcoreMesh(core_axis_name="c", subcore_axis_name="s",
                                    num_cores=NC, num_subcores=NS),
        scratch_types=[
            pltpu.VMEM((CPW, K), jnp.int32),
            pltpu.VMEM((CPW, K), jnp.int32),
            pltpu.VMEM((K, HF), jnp.float32),
            pltpu.VMEM((K, HF), jnp.float32),
            pltpu.VMEM((K, HF), jnp.float32),
            pltpu.VMEM((K, HF), jnp.float32),
            pltpu.VMEM((RPS, HF), jnp.float32),
            pltpu.VMEM_SHARED((NPAD, HF), jnp.float32),
            pltpu.VMEM_SHARED((NPAD, HF), jnp.float32),
            pltpu.SemaphoreType.DMA,
            pltpu.SemaphoreType.DMA,
            pltpu.SemaphoreType.DMA,
            pltpu.SemaphoreType.DMA,
            pltpu.SemaphoreType.DMA,
            pltpu.SemaphoreType.DMA,
            pltpu.SemaphoreType.DMA,
            pltpu.SemaphoreType.DMA,
        ],
        compiler_params=pltpu.CompilerParams(use_tc_tiling_on_sc=False),
    )


BN = 1024
GRID = NPAD // BN


def _tc_init_body(x_ref, win_ref, bin_ref, d0_ref, h_ref, hs0_ref, hs1_ref,
                  dinv_ref):
    xb = x_ref[...]
    h = jnp.maximum(
        jnp.dot(xb, win_ref[...], preferred_element_type=jnp.float32)
        + bin_ref[...], 0.0)
    deg = d0_ref[0, :, 0:1] + d0_ref[1, :, 0:1] + 1.0
    dinv = lax.rsqrt(deg)
    hs = h * dinv
    h_ref[...] = h
    hs0_ref[...] = hs[:, :HF]
    hs1_ref[...] = hs[:, HF:]
    dinv_ref[...] = dinv


def _make_tc_init(interpret=False):
    return pl.pallas_call(
        _tc_init_body,
        grid=(GRID,),
        in_specs=[
            pl.BlockSpec((BN, D_IN), lambda i: (i, 0)),
            pl.BlockSpec((D_IN, DH), lambda i: (0, 0)),
            pl.BlockSpec((1, DH), lambda i: (0, 0)),
            pl.BlockSpec((NC, BN, HF), lambda i: (0, i, 0)),
        ],
        out_specs=[
            pl.BlockSpec((BN, DH), lambda i: (i, 0)),
            pl.BlockSpec((BN, HF), lambda i: (i, 0)),
            pl.BlockSpec((BN, HF), lambda i: (i, 0)),
            pl.BlockSpec((BN, 1), lambda i: (i, 0)),
        ],
        out_shape=[
            jax.ShapeDtypeStruct((NPAD, DH), jnp.float32),
            jax.ShapeDtypeStruct((NPAD, HF), jnp.float32),
            jax.ShapeDtypeStruct((NPAD, HF), jnp.float32),
            jax.ShapeDtypeStruct((NPAD, 1), jnp.float32),
        ],
        interpret=interpret,
    )


def _tc_layer_body(a0_ref, a1_ref, h_ref, h0_ref, dinv_ref, w_ref, g_ref,
                   b_ref, ho_ref, hs0_ref, hs1_ref, *, beta):
    dinv = dinv_ref[...]
    raw = jnp.concatenate([a0_ref[0] + a0_ref[1], a1_ref[0] + a1_ref[1]],
                          axis=1)
    h = h_ref[...]
    agg = dinv * raw + (dinv * dinv) * h
    z = (1.0 - ALPHA) * agg + ALPHA * h0_ref[...]
    z = (1.0 - beta) * z + beta * jnp.dot(
        z, w_ref[0], preferred_element_type=jnp.float32)
    a = jnp.maximum(z, 0.0) + h
    mu = jnp.mean(a, axis=1, keepdims=True)
    var = jnp.mean((a - mu) ** 2, axis=1, keepdims=True)
    hn = (a - mu) * lax.rsqrt(var + 1e-5) * g_ref[0] + b_ref[0]
    hs = hn * dinv
    ho_ref[...] = hn
    hs0_ref[...] = hs[:, :HF]
    hs1_ref[...] = hs[:, HF:]


def _make_tc_layer(li, beta, interpret=False):
    return pl.pallas_call(
        functools.partial(_tc_layer_body, beta=beta),
        grid=(GRID,),
        in_specs=[
            pl.BlockSpec((NC, BN, HF), lambda i: (0, i, 0)),
            pl.BlockSpec((NC, BN, HF), lambda i: (0, i, 0)),
            pl.BlockSpec((BN, DH), lambda i: (i, 0)),
            pl.BlockSpec((BN, DH), lambda i: (i, 0)),
            pl.BlockSpec((BN, 1), lambda i: (i, 0)),
            pl.BlockSpec((1, DH, DH), lambda i, li=li: (li, 0, 0)),
            pl.BlockSpec((1, 1, DH), lambda i, li=li: (li, 0, 0)),
            pl.BlockSpec((1, 1, DH), lambda i, li=li: (li, 0, 0)),
        ],
        out_specs=[
            pl.BlockSpec((BN, DH), lambda i: (i, 0)),
            pl.BlockSpec((BN, HF), lambda i: (i, 0)),
            pl.BlockSpec((BN, HF), lambda i: (i, 0)),
        ],
        out_shape=[
            jax.ShapeDtypeStruct((NPAD, DH), jnp.float32),
            jax.ShapeDtypeStruct((NPAD, HF), jnp.float32),
            jax.ShapeDtypeStruct((NPAD, HF), jnp.float32),
        ],
        interpret=interpret,
    )


def _tc_out_body(h_ref, wout_ref, bout_ref, o_ref):
    o_ref[...] = jnp.dot(
        h_ref[...], wout_ref[...], preferred_element_type=jnp.float32
    ) + bout_ref[...]


def _make_tc_out(interpret=False):
    return pl.pallas_call(
        _tc_out_body,
        grid=(GRID,),
        in_specs=[
            pl.BlockSpec((BN, DH), lambda i: (i, 0)),
            pl.BlockSpec((DH, DOUT), lambda i: (0, 0)),
            pl.BlockSpec((1, DOUT), lambda i: (0, 0)),
        ],
        out_specs=pl.BlockSpec((BN, DOUT), lambda i: (i, 0)),
        out_shape=jax.ShapeDtypeStruct((NPAD, DOUT), jnp.float32),
        interpret=interpret,
    )


_tc_init = _make_tc_init()
_tc_layers = [
    _make_tc_layer(i, float(np.log(THETA / (i + 1) + 1.0))) for i in range(L)
]
_tc_out = _make_tc_out()


def kernel(x, edge_index, W_in, b_in, W_conv, ln_g, ln_b, W_out, b_out):
    row = edge_index[0]
    col = edge_index[1]
    pad = jnp.full((E_PAD - E,), PAD_NODE, jnp.int32)
    row2 = jnp.concatenate([row, pad]).reshape(NW, CPW, K)
    col2 = jnp.concatenate([col, pad]).reshape(NW, CPW, K)

    sc_propagate = _get_sc_propagate()
    deg2 = _get_sc_deg()(col2)

    x_pad = jnp.pad(x, ((0, NPAD - N), (0, 0)))
    h, hs0, hs1, dinv = _tc_init(x_pad, W_in, b_in.reshape(1, DH), deg2)
    h0 = h
    ln_g3 = ln_g.reshape(L, 1, DH)
    ln_b3 = ln_b.reshape(L, 1, DH)
    for i in range(L):
        a0, a1 = sc_propagate(hs0, hs1, row2, col2)
        h, hs0, hs1 = _tc_layers[i](a0, a1, h, h0, dinv, W_conv, ln_g3, ln_b3)
    out = _tc_out(h, W_out, b_out.reshape(1, DOUT))
    return out[:N]
